# baseline (device time: 17265 ns/iter reference)
import jax
import jax.numpy as jnp
from jax import lax
from jax.experimental import pallas as pl
from jax.experimental.pallas import tpu as pltpu

N_DEV = 4


def kernel(x, w_mat):
    m, k_per = x.shape
    _, n = w_mat.shape
    m_per = m // N_DEV
    h = n // 2

    def body(x_ref, w_ref, out_ref, p_ref,
             s_rel_a, s_rel_b, s_dir_a, s_dir_b, s_acc_a, s_acc_b,
             r_rel_a, r_rel_b, r_dir_a, r_dir_b, r_acc_a, r_acc_b,
             send_sems, recv_sems):
        my = lax.axis_index("i")
        left = (my + N_DEV - 1) % N_DEV
        right = (my + 1) % N_DEV

        barrier_sem = pltpu.get_barrier_semaphore()
        for nbr in [left, right]:
            pl.semaphore_signal(
                barrier_sem, inc=1,
                device_id=(nbr,), device_id_type=pl.DeviceIdType.MESH,
            )

        def xrow(c):
            return x_ref[pl.ds(c * m_per, m_per), :].astype(jnp.bfloat16)

        wb = w_ref[:, :].astype(jnp.bfloat16)

        def dot(c, lo, hi):
            return jnp.dot(xrow(c), wb[:, lo:hi],
                           preferred_element_type=jnp.float32)

        g_rel = dot((my + 2) % N_DEV, 0, n)
        s_rel_a[:, :] = g_rel[:, :h].astype(jnp.bfloat16)
        s_rel_b[:, :] = g_rel[:, h:].astype(jnp.bfloat16)
        s_dir_a[:, :] = dot((my + N_DEV - 1) % N_DEV, 0, h).astype(jnp.bfloat16)
        s_dir_b[:, :] = dot((my + 1) % N_DEV, h, n).astype(jnp.bfloat16)

        pl.semaphore_wait(barrier_sem, 2)

        def copy(src, dst, sem_idx, dst_dev):
            return pltpu.make_async_remote_copy(
                src_ref=src, dst_ref=dst,
                send_sem=send_sems.at[sem_idx], recv_sem=recv_sems.at[sem_idx],
                device_id=(dst_dev,), device_id_type=pl.DeviceIdType.MESH,
            )

        rel_a = copy(s_rel_a, r_rel_a, 0, right)
        rel_b = copy(s_rel_b, r_rel_b, 1, left)
        dir_a = copy(s_dir_a, r_dir_a, 2, left)
        dir_b = copy(s_dir_b, r_dir_b, 3, right)
        rel_a.start()
        rel_b.start()
        dir_a.start()
        dir_b.start()

        add_a = dot((my + 1) % N_DEV, 0, h)
        add_b = dot((my + N_DEV - 1) % N_DEV, h, n)

        rel_a.wait_recv()
        s_acc_a[:, :] = (
            r_rel_a[:, :].astype(jnp.float32) + add_a
        ).astype(jnp.bfloat16)
        acc_a = copy(s_acc_a, r_acc_a, 4, right)
        acc_a.start()

        rel_b.wait_recv()
        s_acc_b[:, :] = (
            r_rel_b[:, :].astype(jnp.float32) + add_b
        ).astype(jnp.bfloat16)
        acc_b = copy(s_acc_b, r_acc_b, 5, left)
        acc_b.start()

        p_ref[:, :h] = dot(my, 0, h)
        p_ref[:, h:] = dot(my, h, n)

        dir_a.wait_recv()
        dir_b.wait_recv()
        p_ref[:, :h] = p_ref[:, :h] + r_dir_a[:, :].astype(jnp.float32)
        p_ref[:, h:] = p_ref[:, h:] + r_dir_b[:, :].astype(jnp.float32)

        c0 = 0.7978845608028654
        acc_a.wait_recv()
        y_a = p_ref[:, :h] + r_acc_a[:, :].astype(jnp.float32)
        out_ref[:, :h] = 0.5 * y_a * (
            1.0 + jnp.tanh(c0 * (y_a + 0.044715 * y_a * y_a * y_a)))
        acc_b.wait_recv()
        y_b = p_ref[:, h:] + r_acc_b[:, :].astype(jnp.float32)
        out_ref[:, h:] = 0.5 * y_b * (
            1.0 + jnp.tanh(c0 * (y_b + 0.044715 * y_b * y_b * y_b)))

        for r in (rel_a, rel_b, dir_a, dir_b, acc_a, acc_b):
            r.wait_send()

    half = (m_per, h)
    return pl.pallas_call(
        body,
        out_shape=jax.ShapeDtypeStruct((m_per, n), jnp.float32),
        in_specs=[
            pl.BlockSpec(memory_space=pltpu.VMEM),
            pl.BlockSpec(memory_space=pltpu.VMEM),
        ],
        out_specs=pl.BlockSpec(memory_space=pltpu.VMEM),
        scratch_shapes=(
            [pltpu.VMEM((m_per, n), jnp.float32)]
            + [pltpu.VMEM(half, jnp.bfloat16)] * 12
            + [pltpu.SemaphoreType.DMA((6,)),
               pltpu.SemaphoreType.DMA((6,))]
        ),
        compiler_params=pltpu.CompilerParams(collective_id=0),
    )(x, w_mat)
